# Initial kernel scaffold; baseline (speedup 1.0000x reference)
import jax
import jax.numpy as jnp
from jax.experimental import pallas as pl


def kernel(x, edge_index, W, b, gamma, beta):
    raise NotImplementedError("write your pallas kernel here")


# speedup vs baseline: 3.3746x; 3.3746x over previous
import jax
import jax.numpy as jnp
from jax import lax
from jax.experimental import pallas as pl
from jax.experimental.pallas import tpu as pltpu
from jax.experimental.pallas import tpu_sc as plsc

C = 128
N = 10000
K = 32
NW = 32
NPW = 320
NPAD = NW * NPW
NODES_PER_CHUNK = 4
CHUNK = NODES_PER_CHUNK * K
NCHUNK = NPW // NODES_PER_CHUNK
NVREG = C // 16


def _sc_gather_max(idx3, table):
    mesh = plsc.VectorSubcoreMesh(
        core_axis_name="c", subcore_axis_name="s", num_cores=2, num_subcores=16
    )

    def body(idx_hbm, table_hbm, out_hbm, idx_v, rows0, rows1, out_v, sem0, sem1):
        cid = lax.axis_index("c")
        sid = lax.axis_index("s")
        wid = sid * 2 + cid
        pltpu.sync_copy(idx_hbm.at[wid], idx_v)
        rows = (rows0, rows1)
        sems = (sem0, sem1)
        pltpu.async_copy(table_hbm.at[idx_v.at[0]], rows0, sem0)
        pltpu.async_copy(table_hbm.at[idx_v.at[1]], rows1, sem1)

        def chunk_pair(i, carry):
            for b in range(2):
                c = 2 * i + b
                pltpu.make_async_copy(table_hbm.at[idx_v.at[0]], rows[b], sems[b]).wait()
                for j in range(NODES_PER_CHUNK):
                    r0 = j * K
                    acc = [rows[b][r0, pl.ds(v * 16, 16)] for v in range(NVREG)]
                    for k in range(1, K):
                        for v in range(NVREG):
                            acc[v] = jnp.maximum(
                                acc[v], rows[b][r0 + k, pl.ds(v * 16, 16)]
                            )
                    orow = c * NODES_PER_CHUNK + j
                    for v in range(NVREG):
                        out_v[orow, pl.ds(v * 16, 16)] = acc[v]
                pltpu.async_copy(table_hbm.at[idx_v.at[c + 2]], rows[b], sems[b])
            return carry

        lax.fori_loop(0, NCHUNK // 2, chunk_pair, 0)
        pltpu.make_async_copy(table_hbm.at[idx_v.at[0]], rows0, sem0).wait()
        pltpu.make_async_copy(table_hbm.at[idx_v.at[1]], rows1, sem1).wait()
        pltpu.sync_copy(out_v, out_hbm.at[pl.ds(wid * NPW, NPW)])

    return pl.kernel(
        body,
        out_type=jax.ShapeDtypeStruct((NPAD, C), jnp.float32),
        mesh=mesh,
        scratch_types=[
            pltpu.VMEM((NCHUNK + 2, CHUNK), jnp.int32),
            pltpu.VMEM((CHUNK, C), jnp.float32),
            pltpu.VMEM((CHUNK, C), jnp.float32),
            pltpu.VMEM((NPW, C), jnp.float32),
            pltpu.SemaphoreType.DMA,
            pltpu.SemaphoreType.DMA,
        ],
    )(idx3, table)


def _tc_conv_bn(x_cn, xjm, wd, wb, bvec, gvec, bevec):
    def body(x_ref, xj_ref, wd_ref, wb_ref, b_ref, g_ref, be_ref, o_ref):
        y = jnp.dot(wd_ref[...], x_ref[...], preferred_element_type=jnp.float32)
        y = y + lax.dot_general(
            wb_ref[...], xj_ref[...], (((1,), (1,)), ((), ())),
            preferred_element_type=jnp.float32,
        )
        y = y + b_ref[...]
        valid = lax.broadcasted_iota(jnp.int32, y.shape, 1) < N
        ym = jnp.where(valid, y, 0.0)
        s1 = jnp.sum(ym, axis=1, keepdims=True)
        s2 = jnp.sum(ym * ym, axis=1, keepdims=True)
        mean = s1 * (1.0 / N)
        var = s2 * (1.0 / N) - mean * mean
        inv = lax.rsqrt(var + 1e-5)
        scale = g_ref[...] * inv
        shift = be_ref[...] - mean * scale
        o_ref[...] = jnp.maximum(y * scale + shift, 0.0)

    return pl.pallas_call(
        body,
        out_shape=jax.ShapeDtypeStruct((C, NPAD), jnp.float32),
    )(x_cn, xjm, wd, wb, bvec, gvec, bevec)


def kernel(x, edge_index, W, b, gamma, beta):
    x_cn = x[0, :, :, 0]
    x_cn_pad = jnp.pad(x_cn, ((0, 0), (0, NPAD - N)))
    table = x_cn.T
    e = edge_index[0]
    e_pad = jnp.pad(e, ((0, NPAD - N), (0, 0)))
    idx3 = e_pad.reshape(NW, NCHUNK, CHUNK)
    idx3 = jnp.pad(idx3, ((0, 0), (0, 2), (0, 0)))
    xjm = _sc_gather_max(idx3, table)
    wa = W[:, :C]
    wb = W[:, C:]
    y = _tc_conv_bn(
        x_cn_pad, xjm, wa - wb, wb,
        b.reshape(C, 1), gamma.reshape(C, 1), beta.reshape(C, 1),
    )
    return y[:, :N].reshape(1, C, N, 1)
